# Initial kernel scaffold; baseline (speedup 1.0000x reference)
import jax
import jax.numpy as jnp
from jax.experimental import pallas as pl


def kernel(x, w, b):
    raise NotImplementedError("write your pallas kernel here")


# speedup vs baseline: 38.5333x; 38.5333x over previous
import jax
import jax.numpy as jnp
from jax.experimental import pallas as pl
from jax.experimental.pallas import tpu as pltpu


def _conv_body(H, W, Ho, Wo, Wp, s, kernel):
    KH, KW = kernel

    def body(x_ref, wt_ref, b_ref, o_ref, xp_ref, cols_ref):
        C = x_ref.shape[1]
        L = Ho * Wo
        xt = x_ref[0].T.reshape(H, W, C)
        xp_ref[0] = jnp.zeros((Wp, C), jnp.float32)
        xp_ref[:, 0, :] = jnp.zeros((H + 2, C), jnp.float32)
        xp_ref[1:H + 1, 1:W + 1, :] = xt
        for kh in range(KH):
            for kw in range(KW):
                t = kh * KW + kw
                patch = xp_ref[pl.ds(kh, Ho, s), pl.ds(kw, Wo, s), :]
                cols_ref[:, t * C:(t + 1) * C] = patch.reshape(L, C)
        acc = jnp.dot(cols_ref[...], wt_ref[...],
                      preferred_element_type=jnp.float32)
        acc = acc + b_ref[...]
        o_ref[0] = acc.T.astype(o_ref.dtype)

    return body


def kernel(x, w, b):
    N, C, H, W = x.shape
    Cout, Cin, KH, KW = w.shape
    assert Cin == C
    s, padding = 2, 1
    Ho = (H + 2 * padding - KH) // s + 1
    Wo = (W + 2 * padding - KW) // s + 1
    L = Ho * Wo
    K = C * KH * KW
    Wp = ((W + 2 + 7) // 8) * 8

    xr = x.reshape(N, C, H * W)
    wt = w.transpose(2, 3, 1, 0).reshape(K, Cout)
    brow = b.reshape(1, Cout)

    out = pl.pallas_call(
        _conv_body(H, W, Ho, Wo, Wp, s, (KH, KW)),
        out_shape=jax.ShapeDtypeStruct((N, Cout, L), x.dtype),
        grid=(N,),
        in_specs=[
            pl.BlockSpec((1, C, H * W), lambda n: (n, 0, 0)),
            pl.BlockSpec((K, Cout), lambda n: (0, 0)),
            pl.BlockSpec((1, Cout), lambda n: (0, 0)),
        ],
        out_specs=pl.BlockSpec((1, Cout, L), lambda n: (n, 0, 0)),
        scratch_shapes=[
            pltpu.VMEM((H + 2, Wp, C), jnp.float32),
            pltpu.VMEM((L, K), jnp.float32),
        ],
        compiler_params=pltpu.CompilerParams(
            dimension_semantics=("parallel",)),
    )(xr, wt, brow)

    return out.reshape(N, Cout, Ho, Wo)
